# Initial kernel scaffold; baseline (speedup 1.0000x reference)
import jax
import jax.numpy as jnp
from jax.experimental import pallas as pl


def kernel(X, emb_table, Wx, Wh, b, Wd, bd):
    raise NotImplementedError("write your pallas kernel here")


# speedup vs baseline: 4.9026x; 4.9026x over previous
import jax
import jax.numpy as jnp
from jax.experimental import pallas as pl
from jax.experimental.pallas import tpu as pltpu
from jax.experimental.pallas import tpu_sc as plsc


def _project_table(emb_table, Wx, b):
    V, D = emb_table.shape
    H = Wx.shape[1]
    blk = 4000
    while V % blk:
        blk //= 2
    b2 = b.reshape(1, H)

    def body(e_ref, wx_ref, b_ref, o_ref):
        o_ref[...] = (
            jnp.dot(e_ref[...], wx_ref[...], preferred_element_type=jnp.float32)
            + b_ref[...]
        )

    return pl.pallas_call(
        body,
        grid=(V // blk,),
        in_specs=[
            pl.BlockSpec((blk, D), lambda i: (i, 0)),
            pl.BlockSpec((D, H), lambda i: (0, 0)),
            pl.BlockSpec((1, H), lambda i: (0, 0)),
        ],
        out_specs=pl.BlockSpec((blk, H), lambda i: (i, 0)),
        out_shape=jax.ShapeDtypeStruct((V, H), jnp.float32),
    )(emb_table, Wx, b2)


def _sc_gather(P, idx):
    N = idx.shape[1]
    H = P.shape[1]
    window = 128
    mesh = plsc.VectorSubcoreMesh(core_axis_name="core", subcore_axis_name="subcore")

    @pl.kernel(
        out_type=jax.ShapeDtypeStruct((N, H), P.dtype),
        mesh=mesh,
    )
    def k(p_hbm, i_hbm, o_hbm):
        def body(i_vmem, o_vmem):
            pltpu.sync_copy(p_hbm.at[i_vmem.at[0]], o_vmem)

        pltpu.emit_pipeline(
            body,
            grid=(N // window,),
            in_specs=[pl.BlockSpec((1, window), index_map=lambda i: (0, i))],
            out_specs=[pl.BlockSpec((window, H), index_map=lambda i: (i, 0))],
            core_axis_name=("core", "subcore"),
            dimension_semantics=(pltpu.PARALLEL,),
        )(i_hbm, o_hbm)

    return k(P, idx)


def _rnn(U, Wh, Wd, bd):
    L, B, H = U.shape
    bd2 = bd.reshape(1, 1)

    def body(u_ref, wh_ref, wd_ref, bd_ref, o_ref, h_ref):
        t = pl.program_id(0)

        @pl.when(t == 0)
        def _():
            h_ref[...] = jnp.tanh(u_ref[0])

        @pl.when(t > 0)
        def _():
            h_ref[...] = jnp.tanh(
                u_ref[0]
                + jnp.dot(h_ref[...], wh_ref[...], preferred_element_type=jnp.float32)
            )

        @pl.when(t == L - 1)
        def _():
            logits = (
                jnp.dot(h_ref[...], wd_ref[...], preferred_element_type=jnp.float32)
                + bd_ref[...]
            )
            o_ref[...] = jax.nn.sigmoid(logits)

    return pl.pallas_call(
        body,
        grid=(L,),
        in_specs=[
            pl.BlockSpec((1, B, H), lambda t: (t, 0, 0)),
            pl.BlockSpec((H, H), lambda t: (0, 0)),
            pl.BlockSpec((H, 1), lambda t: (0, 0)),
            pl.BlockSpec((1, 1), lambda t: (0, 0)),
        ],
        out_specs=pl.BlockSpec((B, 1), lambda t: (0, 0)),
        out_shape=jax.ShapeDtypeStruct((B, 1), jnp.float32),
        scratch_shapes=[pltpu.VMEM((B, H), jnp.float32)],
    )(U, Wh, Wd, bd2)


def kernel(X, emb_table, Wx, Wh, b, Wd, bd):
    B, L = X.shape
    H = Wh.shape[0]
    P = _project_table(emb_table, Wx, b)
    idx = X.T.reshape(1, B * L).astype(jnp.int32)
    U = _sc_gather(P, idx).reshape(L, B, H)
    return _rnn(U, Wh, Wd, bd)


# speedup vs baseline: 5.5869x; 1.1396x over previous
import jax
import jax.numpy as jnp
from jax.experimental import pallas as pl
from jax.experimental.pallas import tpu as pltpu
from jax.experimental.pallas import tpu_sc as plsc

_NCHUNKS = 8


def _project_table(emb_table, Wx, b):
    V, D = emb_table.shape
    H = Wx.shape[1]
    blk = 4000
    while V % blk:
        blk //= 2
    b2 = b.reshape(1, H)

    def body(e_ref, wx_ref, b_ref, o_ref):
        acc = jnp.dot(
            e_ref[...].astype(jnp.bfloat16),
            wx_ref[...],
            preferred_element_type=jnp.float32,
        )
        o_ref[...] = acc + b_ref[...]

    return pl.pallas_call(
        body,
        grid=(V // blk,),
        in_specs=[
            pl.BlockSpec((blk, D), lambda i: (i, 0)),
            pl.BlockSpec((D, H), lambda i: (0, 0)),
            pl.BlockSpec((1, H), lambda i: (0, 0)),
        ],
        out_specs=pl.BlockSpec((blk, H), lambda i: (i, 0)),
        out_shape=jax.ShapeDtypeStruct((V, H), jnp.float32),
    )(emb_table, Wx.astype(jnp.bfloat16), b2)


def _sc_gather(P, idx):
    N = idx.shape[1]
    H = P.shape[1]
    window = 128
    mesh = plsc.VectorSubcoreMesh(core_axis_name="core", subcore_axis_name="subcore")

    @pl.kernel(
        out_type=jax.ShapeDtypeStruct((N, H), P.dtype),
        mesh=mesh,
    )
    def k(p_hbm, i_hbm, o_hbm):
        def body(i_vmem, o_vmem):
            pltpu.sync_copy(p_hbm.at[i_vmem.at[0]], o_vmem)

        pltpu.emit_pipeline(
            body,
            grid=(N // window,),
            in_specs=[pl.BlockSpec((1, window), index_map=lambda i: (0, i))],
            out_specs=[pl.BlockSpec((window, H), index_map=lambda i: (i, 0))],
            core_axis_name=("core", "subcore"),
            dimension_semantics=(pltpu.PARALLEL,),
        )(i_hbm, o_hbm)

    return k(P, idx)


def _rnn_chunk(U, h0, Wh16, first):
    Lc, B, H = U.shape

    def body(u_ref, h0_ref, wh_ref, o_ref, h_ref):
        t = pl.program_id(0)

        @pl.when(t == 0)
        def _():
            if first:
                h_ref[...] = jnp.tanh(u_ref[0])
            else:
                h_ref[...] = jnp.tanh(
                    u_ref[0]
                    + jnp.dot(
                        h0_ref[...].astype(jnp.bfloat16),
                        wh_ref[...],
                        preferred_element_type=jnp.float32,
                    )
                )

        @pl.when(t > 0)
        def _():
            h_ref[...] = jnp.tanh(
                u_ref[0]
                + jnp.dot(
                    h_ref[...].astype(jnp.bfloat16),
                    wh_ref[...],
                    preferred_element_type=jnp.float32,
                )
            )

        @pl.when(t == Lc - 1)
        def _():
            o_ref[...] = h_ref[...]

    return pl.pallas_call(
        body,
        grid=(Lc,),
        in_specs=[
            pl.BlockSpec((1, B, H), lambda t: (t, 0, 0)),
            pl.BlockSpec((B, H), lambda t: (0, 0)),
            pl.BlockSpec((H, H), lambda t: (0, 0)),
        ],
        out_specs=pl.BlockSpec((B, H), lambda t: (0, 0)),
        out_shape=jax.ShapeDtypeStruct((B, H), jnp.float32),
        scratch_shapes=[pltpu.VMEM((B, H), jnp.float32)],
    )(U, h0, Wh16)


def _head(h, Wd, bd):
    B, H = h.shape
    bd2 = bd.reshape(1, 1)

    def body(h_ref, wd_ref, bd_ref, o_ref):
        logits = (
            jnp.dot(h_ref[...], wd_ref[...], preferred_element_type=jnp.float32)
            + bd_ref[...]
        )
        o_ref[...] = jax.nn.sigmoid(logits)

    return pl.pallas_call(
        body,
        in_specs=[
            pl.BlockSpec((B, H), lambda: (0, 0)),
            pl.BlockSpec((H, 1), lambda: (0, 0)),
            pl.BlockSpec((1, 1), lambda: (0, 0)),
        ],
        out_specs=pl.BlockSpec((B, 1), lambda: (0, 0)),
        out_shape=jax.ShapeDtypeStruct((B, 1), jnp.float32),
    )(h, Wd, bd2)


def kernel(X, emb_table, Wx, Wh, b, Wd, bd):
    B, L = X.shape
    H = Wh.shape[0]
    nchunks = _NCHUNKS
    while L % nchunks:
        nchunks //= 2
    Lc = L // nchunks

    P = _project_table(emb_table, Wx, b)
    Xt = X.T.astype(jnp.int32)
    Wh16 = Wh.astype(jnp.bfloat16)

    h = jnp.zeros((B, H), dtype=jnp.float32)
    for c in range(nchunks):
        idx = Xt[c * Lc : (c + 1) * Lc].reshape(1, Lc * B)
        U = _sc_gather(P, idx).reshape(Lc, B, H)
        h = _rnn_chunk(U, h, Wh16, first=(c == 0))
    return _head(h, Wd, bd)
